# baseline (speedup 1.0000x reference)
import functools
import math

import jax
import jax.numpy as jnp
from jax import lax
from jax.experimental import pallas as pl
from jax.experimental.pallas import tpu as pltpu
from jax.experimental.pallas import tpu_sc as plsc

_VOCAB = 32000
_PAD = 0
_SMOOTH = 0.1
_CONF = 1.0 - _SMOOTH
_N = 2048
_SMOOTH_VAL = _SMOOTH / (_VOCAB - 2)
_CONST = float(
    (_VOCAB - 2) * _SMOOTH_VAL * math.log(_SMOOTH_VAL)
    + _CONF * math.log(_CONF)
)

_ROW_BLOCK = 128

_NC = 2
_NS = 16
_L = 16
_NW = _NC * _NS
_B_PER_W = _N // _NW


def _tc_dense_block(tgt_ref, x_ref, out_ref):
    i = pl.program_id(0)
    x = x_ref[...]
    tgt = tgt_ref[...]
    valid = tgt != _PAD
    s = jnp.sum(x, axis=1, keepdims=True)
    x0 = x[:, 0:1]
    contrib = _CONST - _SMOOTH_VAL * (s - x0)
    partial = jnp.sum(jnp.where(valid, contrib, 0.0))
    partial = jnp.reshape(partial, (1, 1))

    @pl.when(i == 0)
    def _():
        out_ref[...] = jnp.zeros((1, 1), jnp.float32)

    out_ref[...] += partial


def _sc_gather_body(xf_hbm, tgt_hbm, out_hbm, tgt_v, idx_v, vals_v, acc_v,
                    sem):
    wid = lax.axis_index("s") * _NC + lax.axis_index("c")
    base = wid * _B_PER_W
    pltpu.sync_copy(tgt_hbm.at[pl.ds(base, _B_PER_W)], tgt_v)
    for j in range(_B_PER_W // _L):
        t = tgt_v[pl.ds(j * _L, _L)]
        rows = lax.iota(jnp.int32, _L) + (base + j * _L)
        idx_v[pl.ds(j * _L, _L)] = rows * _VOCAB + t
    pltpu.async_copy(xf_hbm.at[idx_v], vals_v, sem).wait()
    acc = jnp.zeros((_L,), jnp.float32)
    for j in range(_B_PER_W // _L):
        t = tgt_v[pl.ds(j * _L, _L)]
        v = vals_v[pl.ds(j * _L, _L)]
        acc = acc + jnp.where(t != _PAD, v, 0.0)
    acc_v[...] = acc
    pltpu.sync_copy(acc_v, out_hbm.at[wid])


_sc_gather = functools.partial(
    pl.kernel,
    mesh=plsc.VectorSubcoreMesh(core_axis_name="c", subcore_axis_name="s"),
    out_type=jax.ShapeDtypeStruct((_NW, _L), jnp.float32),
    scratch_types=[
        pltpu.VMEM((_B_PER_W,), jnp.int32),
        pltpu.VMEM((_B_PER_W,), jnp.int32),
        pltpu.VMEM((_B_PER_W,), jnp.float32),
        pltpu.VMEM((_L,), jnp.float32),
        pltpu.SemaphoreType.DMA,
    ],
)(_sc_gather_body)


@jax.jit
def kernel(x, target):
    n, v = x.shape
    tgt = target.astype(jnp.int32)
    tgt2d = tgt.reshape(n, 1)

    dense = pl.pallas_call(
        _tc_dense_block,
        grid=(n // _ROW_BLOCK,),
        in_specs=[
            pl.BlockSpec((_ROW_BLOCK, 1), lambda i: (i, 0)),
            pl.BlockSpec((_ROW_BLOCK, v), lambda i: (i, 0)),
        ],
        out_specs=pl.BlockSpec((1, 1), lambda i: (0, 0)),
        out_shape=jax.ShapeDtypeStruct((1, 1), jnp.float32),
    )(tgt2d, x)

    gathered = _sc_gather(x.reshape(-1), tgt)

    return dense[0, 0] + (_SMOOTH_VAL - _CONF) * jnp.sum(gathered)


# speedup vs baseline: 2.6759x; 2.6759x over previous
import functools
import math

import jax
import jax.numpy as jnp
from jax import lax
from jax.experimental import pallas as pl
from jax.experimental.pallas import tpu as pltpu
from jax.experimental.pallas import tpu_sc as plsc

_VOCAB = 32000
_PAD = 0
_SMOOTH = 0.1
_CONF = 1.0 - _SMOOTH
_N = 2048
_SMOOTH_VAL = _SMOOTH / (_VOCAB - 2)
_CONST = float(
    (_VOCAB - 2) * _SMOOTH_VAL * math.log(_SMOOTH_VAL)
    + _CONF * math.log(_CONF)
)

_ROW_BLOCK = 128

_NC = 2
_NS = 16
_L = 16
_NW = _NC * _NS

_N_SC = 512
_N_TC = _N - _N_SC
_ROWS_PER_W = _N_SC // _NW
_CHUNK = 3200
_N_CHUNKS = _VOCAB // _CHUNK
_VREGS = _CHUNK // _L


def _tc_dense_block(tgt_ref, x_ref, out_ref):
    i = pl.program_id(0)
    x = x_ref[...]
    tgt = tgt_ref[...]
    cols = jax.lax.broadcasted_iota(jnp.int32, x.shape, 1)
    valid = tgt != _PAD
    s = jnp.sum(x, axis=1, keepdims=True)
    t = jnp.sum(jnp.where(cols == tgt, x, 0.0), axis=1, keepdims=True)
    x0 = x[:, 0:1]
    contrib = _CONST - _SMOOTH_VAL * (s - x0) + (_SMOOTH_VAL - _CONF) * t
    partial = jnp.sum(jnp.where(valid, contrib, 0.0))
    partial = jnp.reshape(partial, (1, 1))

    @pl.when(i == 0)
    def _():
        out_ref[...] = jnp.zeros((1, 1), jnp.float32)

    out_ref[...] += partial


def _sc_slab_body(x_hbm, tgt_hbm, out_hbm, buf0, buf1, tgt_v, acc_v,
                  sem0, sem1, semt):
    wid = lax.axis_index("s") * _NC + lax.axis_index("c")
    base = _N_TC + wid * _ROWS_PER_W

    pltpu.async_copy(tgt_hbm.at[pl.ds(base, _ROWS_PER_W)], tgt_v, semt).wait()
    tgt = tgt_v[...]
    validf = jnp.where(tgt != _PAD, 1.0, 0.0)
    lanes = lax.iota(jnp.int32, _L)

    t_s = [tgt[r] for r in range(_ROWS_PER_W)]
    wsplat = []
    for r in range(_ROWS_PER_W):
        w_r = jnp.where(t_s[r] != _PAD, jnp.float32(-_SMOOTH_VAL),
                        jnp.float32(0.0))
        wsplat.append(jnp.broadcast_to(w_r, (_L,)))

    bufs = (buf0, buf1)
    sems = (sem0, sem1)

    def start(j):
        return pltpu.async_copy(
            x_hbm.at[pl.ds(base, _ROWS_PER_W), pl.ds(j * _CHUNK, _CHUNK)],
            bufs[j % 2],
            sems[j % 2],
        )

    extra = jnp.float32(_CONST) * validf
    acc = (jnp.zeros((_L,), jnp.float32),) * 4

    copy = start(0)
    for j in range(_N_CHUNKS):
        nxt = start(j + 1) if j + 1 < _N_CHUNKS else None
        copy.wait()
        buf = bufs[j % 2]

        def body(k, carry):
            a0, a1, a2, a3 = carry
            off = k * _L
            for r in range(_ROWS_PER_W):
                v = buf[r, pl.ds(off, _L)] * wsplat[r]
                if r % 4 == 0:
                    a0 = a0 + v
                elif r % 4 == 1:
                    a1 = a1 + v
                elif r % 4 == 2:
                    a2 = a2 + v
                else:
                    a3 = a3 + v
            return (a0, a1, a2, a3)

        acc = lax.fori_loop(0, _VREGS, body, acc, unroll=False)

        c0 = j * _CHUNK
        for r in range(_ROWS_PER_W):
            if j == 0:
                vec0 = buf[r, pl.ds(0, _L)]
                w0 = jnp.where(t_s[r] != _PAD, jnp.float32(_SMOOTH_VAL),
                               jnp.float32(0.0))
                extra = extra + vec0 * jnp.where(lanes == 0, w0, 0.0)
            pos = t_s[r] - c0
            inb = (pos >= 0) & (pos < _CHUNK) & (t_s[r] != _PAD)
            wc = jnp.where(inb, jnp.float32(_SMOOTH_VAL - _CONF),
                           jnp.float32(0.0))
            lo = pl.multiple_of(jnp.clip(pos - pos % _L, 0, _CHUNK - _L), _L)
            vec = buf[r, pl.ds(lo, _L)]
            extra = extra + vec * jnp.where(lanes == pos - lo, wc, 0.0)

        copy = nxt

    total = acc[0] + acc[1] + acc[2] + acc[3] + extra
    acc_v[...] = total
    pltpu.sync_copy(acc_v, out_hbm.at[wid])


_sc_slab = functools.partial(
    pl.kernel,
    mesh=plsc.VectorSubcoreMesh(core_axis_name="c", subcore_axis_name="s"),
    out_type=jax.ShapeDtypeStruct((_NW, _L), jnp.float32),
    scratch_types=[
        pltpu.VMEM((_ROWS_PER_W, _CHUNK), jnp.float32),
        pltpu.VMEM((_ROWS_PER_W, _CHUNK), jnp.float32),
        pltpu.VMEM((_ROWS_PER_W,), jnp.int32),
        pltpu.VMEM((_L,), jnp.float32),
        pltpu.SemaphoreType.DMA,
        pltpu.SemaphoreType.DMA,
        pltpu.SemaphoreType.DMA,
    ],
    compiler_params=pltpu.CompilerParams(use_tc_tiling_on_sc=True),
)(_sc_slab_body)


@jax.jit
def kernel(x, target):
    n, v = x.shape
    tgt = target.astype(jnp.int32)
    tgt2d = tgt[:_N_TC].reshape(_N_TC, 1)

    dense = pl.pallas_call(
        _tc_dense_block,
        grid=(_N_TC // _ROW_BLOCK,),
        in_specs=[
            pl.BlockSpec((_ROW_BLOCK, 1), lambda i: (i, 0)),
            pl.BlockSpec((_ROW_BLOCK, v), lambda i: (i, 0)),
        ],
        out_specs=pl.BlockSpec((1, 1), lambda i: (0, 0)),
        out_shape=jax.ShapeDtypeStruct((1, 1), jnp.float32),
    )(tgt2d, x)

    sc_part = _sc_slab(x, tgt)

    return dense[0, 0] + jnp.sum(sc_part)


# speedup vs baseline: 2.7079x; 1.0119x over previous
import functools
import math

import jax
import jax.numpy as jnp
from jax import lax
from jax.experimental import pallas as pl
from jax.experimental.pallas import tpu as pltpu
from jax.experimental.pallas import tpu_sc as plsc

_VOCAB = 32000
_PAD = 0
_SMOOTH = 0.1
_CONF = 1.0 - _SMOOTH
_N = 2048
_SMOOTH_VAL = _SMOOTH / (_VOCAB - 2)
_CONST = float(
    (_VOCAB - 2) * _SMOOTH_VAL * math.log(_SMOOTH_VAL)
    + _CONF * math.log(_CONF)
)

_ROW_BLOCK = 128

_NC = 2
_NS = 16
_L = 16
_NW = _NC * _NS

_N_SC = 256
_N_TC = _N - _N_SC
_GROUP_ROWS = 16
_HALF = _VOCAB // 2
_CHUNK = 3200
_N_CHUNKS = _HALF // _CHUNK
_VREGS = _CHUNK // _L


def _tc_dense_block(tgt_ref, x_ref, out_ref):
    i = pl.program_id(0)
    x = x_ref[...]
    tgt = tgt_ref[...]
    cols = jax.lax.broadcasted_iota(jnp.int32, x.shape, 1)
    valid = tgt != _PAD
    s = jnp.sum(x, axis=1, keepdims=True)
    t = jnp.sum(jnp.where(cols == tgt, x, 0.0), axis=1, keepdims=True)
    x0 = x[:, 0:1]
    contrib = _CONST - _SMOOTH_VAL * (s - x0) + (_SMOOTH_VAL - _CONF) * t
    partial = jnp.sum(jnp.where(valid, contrib, 0.0))
    partial = jnp.reshape(partial, (1, 1))

    @pl.when(i == 0)
    def _():
        out_ref[...] = jnp.zeros((1, 1), jnp.float32)

    out_ref[...] += partial


def _sc_slab_body(x_hbm, tgt_hbm, out_hbm, buf0, buf1, tgt_v, acc_v,
                  sem0, sem1, semt):
    wid = lax.axis_index("s") * _NC + lax.axis_index("c")
    group = wid // 2
    half = wid % 2
    base = _N_TC + group * _GROUP_ROWS
    c_base = half * _HALF

    pltpu.async_copy(tgt_hbm.at[pl.ds(base, _GROUP_ROWS)], tgt_v, semt).wait()
    tgt = tgt_v[...]
    validf = jnp.where(tgt != _PAD, 1.0, 0.0)
    lanes = lax.iota(jnp.int32, _L)

    t_s = [tgt[r] for r in range(_GROUP_ROWS)]
    wsplat = []
    for r in range(_GROUP_ROWS):
        w_r = jnp.where(t_s[r] != _PAD, jnp.float32(-_SMOOTH_VAL),
                        jnp.float32(0.0))
        wsplat.append(jnp.broadcast_to(w_r, (_L,)))

    bufs = (buf0, buf1)
    sems = (sem0, sem1)

    def start(j):
        off = pl.multiple_of(c_base + j * _CHUNK, 128)
        return pltpu.async_copy(
            x_hbm.at[pl.ds(base, _GROUP_ROWS), pl.ds(off, _CHUNK)],
            bufs[j % 2],
            sems[j % 2],
        )

    cscale = jnp.where(half == 0, jnp.float32(_CONST), jnp.float32(0.0))
    extra = cscale * validf
    acc = (jnp.zeros((_L,), jnp.float32),) * _GROUP_ROWS

    copy = start(0)
    for j in range(_N_CHUNKS):
        nxt = start(j + 1) if j + 1 < _N_CHUNKS else None
        copy.wait()
        buf = bufs[j % 2]

        def body(k, carry):
            out = list(carry)
            for u in range(2):
                off = (2 * k + u) * _L
                for r in range(_GROUP_ROWS):
                    out[r] = out[r] + buf[r, pl.ds(off, _L)]
            return tuple(out)

        acc = lax.fori_loop(0, _VREGS // 2, body, acc)

        for r in range(_GROUP_ROWS):
            if j == 0:
                vec0 = buf[r, pl.ds(0, _L)]
                w0 = jnp.where((half == 0) & (t_s[r] != _PAD),
                               jnp.float32(_SMOOTH_VAL), jnp.float32(0.0))
                extra = extra + vec0 * jnp.where(lanes == 0, w0, 0.0)
            pos = t_s[r] - (c_base + j * _CHUNK)
            inb = (pos >= 0) & (pos < _CHUNK) & (t_s[r] != _PAD)
            wc = jnp.where(inb, jnp.float32(_SMOOTH_VAL - _CONF),
                           jnp.float32(0.0))
            lo = pl.multiple_of(jnp.clip(pos - pos % _L, 0, _CHUNK - _L), _L)
            vec = buf[r, pl.ds(lo, _L)]
            extra = extra + vec * jnp.where(lanes == pos - lo, wc, 0.0)

        copy = nxt

    total = extra
    for r in range(_GROUP_ROWS):
        total = total + acc[r] * wsplat[r]
    acc_v[...] = total
    pltpu.sync_copy(acc_v, out_hbm.at[wid])


_sc_slab = functools.partial(
    pl.kernel,
    mesh=plsc.VectorSubcoreMesh(core_axis_name="c", subcore_axis_name="s"),
    out_type=jax.ShapeDtypeStruct((_NW, _L), jnp.float32),
    scratch_types=[
        pltpu.VMEM((_GROUP_ROWS, _CHUNK), jnp.float32),
        pltpu.VMEM((_GROUP_ROWS, _CHUNK), jnp.float32),
        pltpu.VMEM((_GROUP_ROWS,), jnp.int32),
        pltpu.VMEM((_L,), jnp.float32),
        pltpu.SemaphoreType.DMA,
        pltpu.SemaphoreType.DMA,
        pltpu.SemaphoreType.DMA,
    ],
    compiler_params=pltpu.CompilerParams(use_tc_tiling_on_sc=True),
)(_sc_slab_body)


@jax.jit
def kernel(x, target):
    n, v = x.shape
    tgt = target.astype(jnp.int32)
    tgt2d = tgt[:_N_TC].reshape(_N_TC, 1)

    dense = pl.pallas_call(
        _tc_dense_block,
        grid=(_N_TC // _ROW_BLOCK,),
        in_specs=[
            pl.BlockSpec((_ROW_BLOCK, 1), lambda i: (i, 0)),
            pl.BlockSpec((_ROW_BLOCK, v), lambda i: (i, 0)),
        ],
        out_specs=pl.BlockSpec((1, 1), lambda i: (0, 0)),
        out_shape=jax.ShapeDtypeStruct((1, 1), jnp.float32),
    )(tgt2d, x)

    sc_part = _sc_slab(x, tgt)

    return dense[0, 0] + jnp.sum(sc_part)


# speedup vs baseline: 3.3625x; 1.2417x over previous
import math

import jax
import jax.numpy as jnp
from jax.experimental import pallas as pl

_VOCAB = 32000
_PAD = 0
_SMOOTH = 0.1
_CONF = 1.0 - _SMOOTH
_N = 2048
_SMOOTH_VAL = _SMOOTH / (_VOCAB - 2)
_CONST = float(
    (_VOCAB - 2) * _SMOOTH_VAL * math.log(_SMOOTH_VAL)
    + _CONF * math.log(_CONF)
)

_ROW_BLOCK = 128


def _loss_block(tgt_ref, x_ref, out_ref):
    i = pl.program_id(0)
    x = x_ref[...]
    tgt = tgt_ref[...]
    cols = jax.lax.broadcasted_iota(jnp.int32, x.shape, 1)
    valid = tgt != _PAD
    s = jnp.sum(x, axis=1, keepdims=True)
    t = jnp.sum(jnp.where(cols == tgt, x, 0.0), axis=1, keepdims=True)
    x0 = x[:, 0:1]
    contrib = _CONST - _SMOOTH_VAL * (s - x0) + (_SMOOTH_VAL - _CONF) * t
    partial = jnp.sum(jnp.where(valid, contrib, 0.0))
    partial = jnp.reshape(partial, (1, 1))

    @pl.when(i == 0)
    def _():
        out_ref[...] = jnp.zeros((1, 1), jnp.float32)

    out_ref[...] += partial


@jax.jit
def kernel(x, target):
    n, v = x.shape
    tgt2d = target.astype(jnp.int32).reshape(n, 1)
    out = pl.pallas_call(
        _loss_block,
        grid=(n // _ROW_BLOCK,),
        in_specs=[
            pl.BlockSpec((_ROW_BLOCK, 1), lambda i: (i, 0)),
            pl.BlockSpec((_ROW_BLOCK, v), lambda i: (i, 0)),
        ],
        out_specs=pl.BlockSpec((1, 1), lambda i: (0, 0)),
        out_shape=jax.ShapeDtypeStruct((1, 1), jnp.float32),
    )(tgt2d, x)
    return out[0, 0]
